# baseline (speedup 1.0000x reference)
import functools

import jax
import jax.numpy as jnp
from jax import lax
from jax.experimental import pallas as pl
from jax.experimental.pallas import tpu as pltpu
from jax.experimental.pallas import tpu_sc as plsc

N_NODES = 10000
NPAD = 10240
D_FEAT = 128
N_EDGES = 320000
NW = 32
CH_T = 80
E_PAD = NW * CH_T * 128
PAD_ROW = NPAD - 1
ROWS_T = NPAD // 16

_mesh = plsc.VectorSubcoreMesh(core_axis_name="c", subcore_axis_name="s")


def _deg_body(dst_hbm, ones_hbm, zeros_hbm, out_hbm, idx_v, ones_v, zer_v, deg_sh):
    c = lax.axis_index("c")
    s = lax.axis_index("s")
    wid = s * 2 + c
    pltpu.sync_copy(ones_hbm, ones_v)
    pltpu.sync_copy(zeros_hbm, zer_v)
    pltpu.sync_copy(zer_v, deg_sh.at[pl.ds(s * ROWS_T, ROWS_T)])
    pltpu.sync_copy(dst_hbm.at[wid], idx_v)
    plsc.subcore_barrier()

    def body(j, carry):
        pltpu.sync_copy(ones_v, deg_sh.at[idx_v.at[j]], add=True)
        return carry

    lax.fori_loop(0, CH_T, body, 0)
    plsc.subcore_barrier()
    pltpu.sync_copy(deg_sh.at[pl.ds(s * ROWS_T, ROWS_T)],
                    out_hbm.at[c, pl.ds(s * ROWS_T, ROWS_T)])


def _deg_sc(dstp, ones1, zeros1):
    return pl.kernel(
        _deg_body,
        out_type=jax.ShapeDtypeStruct((2, NPAD), jnp.float32),
        mesh=_mesh,
        scratch_types=[
            pltpu.VMEM((CH_T, 128), jnp.int32),
            pltpu.VMEM((128,), jnp.float32),
            pltpu.VMEM((ROWS_T,), jnp.float32),
            pltpu.VMEM_SHARED((NPAD,), jnp.float32),
        ],
    )(dstp, ones1, zeros1)


NB = 2
NGRP = 4
K_C0 = 4
K_C1 = 16
KMAX = 16


def _edge_loop(src_hbm, dst_hbm, h_hbm, agg_sh,
               sidx, didx, rows, gsem, ssem, s, k):
    def group(g, carry):
        for b in range(NB):
            pltpu.sync_copy(src_hbm.at[s, b, g], sidx[b].at[pl.ds(0, k)])
            pltpu.sync_copy(dst_hbm.at[s, b, g], didx[b].at[pl.ds(0, k)])
            pltpu.async_copy(h_hbm.at[sidx[b].at[0]], rows[b], gsem[b])

        def body(i, carry2):
            scat = []
            for b in range(NB):
                pltpu.make_async_copy(h_hbm.at[sidx[b].at[i]], rows[b],
                                      gsem[b]).wait()
                scat.append(pltpu.async_copy(rows[b],
                                             agg_sh.at[didx[b].at[i]],
                                             ssem[b], add=True))
            for b in range(NB):
                scat[b].wait()
                jn = jnp.minimum(i + 1, k - 1)
                pltpu.async_copy(h_hbm.at[sidx[b].at[jn]], rows[b], gsem[b])
            return carry2

        lax.fori_loop(0, k, body, 0)
        for b in range(NB):
            pltpu.make_async_copy(h_hbm.at[sidx[b].at[k - 1]], rows[b],
                                  gsem[b]).wait()
        return carry

    lax.fori_loop(0, NGRP, group, 0)


def _agg_body(src0_hbm, dst0_hbm, src1_hbm, dst1_hbm, h_hbm, zeros_hbm,
              out_hbm, sidx0, sidx1, didx0, didx1, rows0, rows1, agg_sh,
              gsem0, gsem1, ssem0, ssem1):
    c = lax.axis_index("c")
    s = lax.axis_index("s")
    sidx = (sidx0, sidx1)
    didx = (didx0, didx1)
    rows = (rows0, rows1)
    gsem = (gsem0, gsem1)
    ssem = (ssem0, ssem1)
    pltpu.sync_copy(zeros_hbm, agg_sh.at[pl.ds(s * ROWS_T, ROWS_T)])
    plsc.subcore_barrier()

    @pl.when(c == 0)
    def _():
        _edge_loop(src0_hbm, dst0_hbm, h_hbm, agg_sh,
                   sidx, didx, rows, gsem, ssem, s, K_C0)

    @pl.when(c == 1)
    def _():
        _edge_loop(src1_hbm, dst1_hbm, h_hbm, agg_sh,
                   sidx, didx, rows, gsem, ssem, s, K_C1)

    plsc.subcore_barrier()
    pltpu.sync_copy(agg_sh.at[pl.ds(s * ROWS_T, ROWS_T)],
                    out_hbm.at[c, pl.ds(s * ROWS_T, ROWS_T)])


def _agg_sc(src0, dst0, src1, dst1, h, zeros2):
    return pl.kernel(
        _agg_body,
        out_type=jax.ShapeDtypeStruct((2, NPAD, D_FEAT), jnp.float32),
        mesh=_mesh,
        scratch_types=[
            pltpu.VMEM((KMAX, 128), jnp.int32),
            pltpu.VMEM((KMAX, 128), jnp.int32),
            pltpu.VMEM((KMAX, 128), jnp.int32),
            pltpu.VMEM((KMAX, 128), jnp.int32),
            pltpu.VMEM((128, D_FEAT), jnp.float32),
            pltpu.VMEM((128, D_FEAT), jnp.float32),
            pltpu.VMEM_SHARED((NPAD, D_FEAT), jnp.float32),
            pltpu.SemaphoreType.DMA,
            pltpu.SemaphoreType.DMA,
            pltpu.SemaphoreType.DMA,
            pltpu.SemaphoreType.DMA,
        ],
    )(src0, dst0, src1, dst1, h, zeros2)


def _stage1_body(x_ref, degp_ref, w_ref, h_ref, dinv_ref):
    deg = jnp.sum(degp_ref[...], axis=1, keepdims=True) + 1.0
    dinv = lax.rsqrt(deg)
    dinv_ref[...] = dinv
    h_ref[...] = jnp.dot(x_ref[...], w_ref[...],
                         preferred_element_type=jnp.float32) * dinv


def _stage1_tc(x_pad, degp_t, w1):
    return pl.pallas_call(
        _stage1_body,
        out_shape=(
            jax.ShapeDtypeStruct((NPAD, D_FEAT), jnp.float32),
            jax.ShapeDtypeStruct((NPAD, 1), jnp.float32),
        ),
    )(x_pad, degp_t, w1)


def _stage2_body(agga_ref, aggb_ref, hp_ref, dinv_ref, b_ref, w_ref, out_ref):
    dinv = dinv_ref[...]
    t = (agga_ref[...] + aggb_ref[...] + hp_ref[...]) * dinv + b_ref[...]
    h = jnp.maximum(t, 0.0)
    out_ref[...] = jnp.dot(h, w_ref[...],
                           preferred_element_type=jnp.float32) * dinv


def _stage2_tc(agga, aggb, hp, dinv, b1, w2):
    return pl.pallas_call(
        _stage2_body,
        out_shape=jax.ShapeDtypeStruct((NPAD, D_FEAT), jnp.float32),
    )(agga, aggb, hp, dinv, b1, w2)


def _stage3_body(agga_ref, aggb_ref, hp_ref, dinv_ref, b_ref, w_ref,
                 bmask_ref, out_ref):
    t = (agga_ref[...] + aggb_ref[...] + hp_ref[...]) * dinv_ref[...] + b_ref[...]
    h = jnp.maximum(t, 0.0)
    logits = jnp.dot(h, w_ref[...],
                     preferred_element_type=jnp.float32) + bmask_ref[...]
    m = jnp.max(logits, axis=1, keepdims=True)
    e = jnp.exp(logits - m)
    out_ref[...] = e / jnp.sum(e, axis=1, keepdims=True)


def _stage3_tc(agga, aggb, hp, dinv, b2, wfc_pad, bmask):
    return pl.pallas_call(
        _stage3_body,
        out_shape=jax.ShapeDtypeStruct((NPAD, 128), jnp.float32),
    )(agga, aggb, hp, dinv, b2, wfc_pad, bmask)


def kernel(x, edge_index, W1, b1, W2, b2, Wfc, bfc):
    x_pad = jnp.pad(x, ((0, NPAD - N_NODES), (0, 0)))
    epad = jnp.full((E_PAD - N_EDGES,), PAD_ROW, jnp.int32)

    n0 = 16 * NB * NGRP * K_C0 * 128

    def _prep(idx):
        a = jnp.concatenate([idx, epad])
        a0 = a[:n0].reshape(16, NGRP, K_C0, NB, 128).transpose(0, 3, 1, 2, 4)
        a1 = a[n0:].reshape(16, NGRP, K_C1, NB, 128).transpose(0, 3, 1, 2, 4)
        return a0, a1

    src0, src1 = _prep(edge_index[0])
    dst0, dst1 = _prep(edge_index[1])
    dstp_flat = jnp.concatenate([edge_index[1], epad]).reshape(NW, CH_T, 128)

    ones1 = jnp.ones((128,), jnp.float32)
    zeros1 = jnp.zeros((ROWS_T,), jnp.float32)
    zeros2 = jnp.zeros((ROWS_T, D_FEAT), jnp.float32)
    b1r = b1.reshape(1, D_FEAT)
    b2r = b2.reshape(1, D_FEAT)
    wfc_pad = jnp.pad(Wfc, ((0, 0), (0, 128 - Wfc.shape[1])))
    bmask = jnp.concatenate(
        [bfc, jnp.full((128 - bfc.shape[0],), -1e30, jnp.float32)]).reshape(1, 128)

    degp = _deg_sc(dstp_flat, ones1, zeros1)
    degp_t = degp.T

    h1p, dinv = _stage1_tc(x_pad, degp_t, W1)
    agg1 = _agg_sc(src0, dst0, src1, dst1, h1p, zeros2)
    h2p = _stage2_tc(agg1[0], agg1[1], h1p, dinv, b1r, W2)
    agg2 = _agg_sc(src0, dst0, src1, dst1, h2p, zeros2)
    probs = _stage3_tc(agg2[0], agg2[1], h2p, dinv, b2r, wfc_pad, bmask)
    return probs[:N_NODES, :16]


# speedup vs baseline: 1.1561x; 1.1561x over previous
import functools

import jax
import jax.numpy as jnp
from jax import lax
from jax.experimental import pallas as pl
from jax.experimental.pallas import tpu as pltpu
from jax.experimental.pallas import tpu_sc as plsc

N_NODES = 10000
NPAD = 10240
D_FEAT = 128
N_EDGES = 320000
NW = 32
CH_T = 80
E_PAD = NW * CH_T * 128
PAD_ROW = NPAD - 1
ROWS_T = NPAD // 16

_mesh = plsc.VectorSubcoreMesh(core_axis_name="c", subcore_axis_name="s")


def _deg_body(dst_hbm, ones_hbm, zeros_hbm, out_hbm, idx_v, ones_v, zer_v, deg_sh):
    c = lax.axis_index("c")
    s = lax.axis_index("s")
    wid = s * 2 + c
    pltpu.sync_copy(ones_hbm, ones_v)
    pltpu.sync_copy(zeros_hbm, zer_v)
    pltpu.sync_copy(zer_v, deg_sh.at[pl.ds(s * ROWS_T, ROWS_T)])
    pltpu.sync_copy(dst_hbm.at[wid], idx_v)
    plsc.subcore_barrier()

    def body(j, carry):
        pltpu.sync_copy(ones_v, deg_sh.at[idx_v.at[j]], add=True)
        return carry

    lax.fori_loop(0, CH_T, body, 0)
    plsc.subcore_barrier()
    pltpu.sync_copy(deg_sh.at[pl.ds(s * ROWS_T, ROWS_T)],
                    out_hbm.at[c, pl.ds(s * ROWS_T, ROWS_T)])


def _deg_sc(dstp, ones1, zeros1):
    return pl.kernel(
        _deg_body,
        out_type=jax.ShapeDtypeStruct((2, NPAD), jnp.float32),
        mesh=_mesh,
        scratch_types=[
            pltpu.VMEM((CH_T, 128), jnp.int32),
            pltpu.VMEM((128,), jnp.float32),
            pltpu.VMEM((ROWS_T,), jnp.float32),
            pltpu.VMEM_SHARED((NPAD,), jnp.float32),
        ],
    )(dstp, ones1, zeros1)


NB = 2
NGRP = 2
GC = CH_T // NB // NGRP


def _agg_body(src_hbm, dst_hbm, h_hbm, zeros_hbm, out_hbm,
              sidx0, sidx1, didx0, didx1, rows0, rows1, agg_sh,
              gsem0, gsem1, ssem0, ssem1):
    c = lax.axis_index("c")
    s = lax.axis_index("s")
    wid = s * 2 + c
    sidx = (sidx0, sidx1)
    didx = (didx0, didx1)
    rows = (rows0, rows1)
    gsem = (gsem0, gsem1)
    ssem = (ssem0, ssem1)
    pltpu.sync_copy(zeros_hbm, agg_sh.at[pl.ds(s * ROWS_T, ROWS_T)])
    plsc.subcore_barrier()

    def group(g, carry):
        for b in range(NB):
            pltpu.sync_copy(src_hbm.at[wid, b, g], sidx[b])
            pltpu.sync_copy(dst_hbm.at[wid, b, g], didx[b])
            pltpu.async_copy(h_hbm.at[sidx[b].at[0]], rows[b], gsem[b])

        def body(i, carry2):
            scat = []
            for b in range(NB):
                pltpu.make_async_copy(h_hbm.at[sidx[b].at[i]], rows[b],
                                      gsem[b]).wait()
                scat.append(pltpu.async_copy(rows[b],
                                             agg_sh.at[didx[b].at[i]],
                                             ssem[b], add=True))
            for b in range(NB):
                scat[b].wait()
                jn = jnp.minimum(i + 1, GC - 1)
                pltpu.async_copy(h_hbm.at[sidx[b].at[jn]], rows[b], gsem[b])
            return carry2

        lax.fori_loop(0, GC, body, 0)
        for b in range(NB):
            pltpu.make_async_copy(h_hbm.at[sidx[b].at[GC - 1]], rows[b],
                                  gsem[b]).wait()
        return carry

    lax.fori_loop(0, NGRP, group, 0)
    plsc.subcore_barrier()
    pltpu.sync_copy(agg_sh.at[pl.ds(s * ROWS_T, ROWS_T)],
                    out_hbm.at[c, pl.ds(s * ROWS_T, ROWS_T)])


def _agg_sc(srcp, dstp, h, zeros2):
    return pl.kernel(
        _agg_body,
        out_type=jax.ShapeDtypeStruct((2, NPAD, D_FEAT), jnp.float32),
        mesh=_mesh,
        scratch_types=[
            pltpu.VMEM((GC, 128), jnp.int32),
            pltpu.VMEM((GC, 128), jnp.int32),
            pltpu.VMEM((GC, 128), jnp.int32),
            pltpu.VMEM((GC, 128), jnp.int32),
            pltpu.VMEM((128, D_FEAT), jnp.float32),
            pltpu.VMEM((128, D_FEAT), jnp.float32),
            pltpu.VMEM_SHARED((NPAD, D_FEAT), jnp.float32),
            pltpu.SemaphoreType.DMA,
            pltpu.SemaphoreType.DMA,
            pltpu.SemaphoreType.DMA,
            pltpu.SemaphoreType.DMA,
        ],
    )(srcp, dstp, h, zeros2)


def _stage1_body(x_ref, degp_ref, w_ref, h_ref, dinv_ref):
    deg = jnp.sum(degp_ref[...], axis=1, keepdims=True) + 1.0
    dinv = lax.rsqrt(deg)
    dinv_ref[...] = dinv
    h_ref[...] = jnp.dot(x_ref[...], w_ref[...],
                         preferred_element_type=jnp.float32) * dinv


def _stage1_tc(x_pad, degp_t, w1):
    return pl.pallas_call(
        _stage1_body,
        out_shape=(
            jax.ShapeDtypeStruct((NPAD, D_FEAT), jnp.float32),
            jax.ShapeDtypeStruct((NPAD, 1), jnp.float32),
        ),
    )(x_pad, degp_t, w1)


def _stage2_body(agga_ref, aggb_ref, hp_ref, dinv_ref, b_ref, w_ref, out_ref):
    dinv = dinv_ref[...]
    t = (agga_ref[...] + aggb_ref[...] + hp_ref[...]) * dinv + b_ref[...]
    h = jnp.maximum(t, 0.0)
    out_ref[...] = jnp.dot(h, w_ref[...],
                           preferred_element_type=jnp.float32) * dinv


def _stage2_tc(agga, aggb, hp, dinv, b1, w2):
    return pl.pallas_call(
        _stage2_body,
        out_shape=jax.ShapeDtypeStruct((NPAD, D_FEAT), jnp.float32),
    )(agga, aggb, hp, dinv, b1, w2)


def _stage3_body(agga_ref, aggb_ref, hp_ref, dinv_ref, b_ref, w_ref,
                 bmask_ref, out_ref):
    t = (agga_ref[...] + aggb_ref[...] + hp_ref[...]) * dinv_ref[...] + b_ref[...]
    h = jnp.maximum(t, 0.0)
    logits = jnp.dot(h, w_ref[...],
                     preferred_element_type=jnp.float32) + bmask_ref[...]
    m = jnp.max(logits, axis=1, keepdims=True)
    e = jnp.exp(logits - m)
    out_ref[...] = e / jnp.sum(e, axis=1, keepdims=True)


def _stage3_tc(agga, aggb, hp, dinv, b2, wfc_pad, bmask):
    return pl.pallas_call(
        _stage3_body,
        out_shape=jax.ShapeDtypeStruct((NPAD, 128), jnp.float32),
    )(agga, aggb, hp, dinv, b2, wfc_pad, bmask)


def kernel(x, edge_index, W1, b1, W2, b2, Wfc, bfc):
    x_pad = jnp.pad(x, ((0, NPAD - N_NODES), (0, 0)))
    epad = jnp.full((E_PAD - N_EDGES,), PAD_ROW, jnp.int32)

    def _prep(idx):
        a = jnp.concatenate([idx, epad]).reshape(NW, NGRP, GC, NB, 128)
        return a.transpose(0, 3, 1, 2, 4)

    srcp = _prep(edge_index[0])
    dstp = _prep(edge_index[1])
    dstp_flat = jnp.concatenate([edge_index[1], epad]).reshape(NW, CH_T, 128)

    ones1 = jnp.ones((128,), jnp.float32)
    zeros1 = jnp.zeros((ROWS_T,), jnp.float32)
    zeros2 = jnp.zeros((ROWS_T, D_FEAT), jnp.float32)
    b1r = b1.reshape(1, D_FEAT)
    b2r = b2.reshape(1, D_FEAT)
    wfc_pad = jnp.pad(Wfc, ((0, 0), (0, 128 - Wfc.shape[1])))
    bmask = jnp.concatenate(
        [bfc, jnp.full((128 - bfc.shape[0],), -1e30, jnp.float32)]).reshape(1, 128)

    degp = _deg_sc(dstp_flat, ones1, zeros1)
    degp_t = degp.T

    h1p, dinv = _stage1_tc(x_pad, degp_t, W1)
    agg1 = _agg_sc(srcp, dstp, h1p, zeros2)
    h2p = _stage2_tc(agg1[0], agg1[1], h1p, dinv, b1r, W2)
    agg2 = _agg_sc(srcp, dstp, h2p, zeros2)
    probs = _stage3_tc(agg2[0], agg2[1], h2p, dinv, b2r, wfc_pad, bmask)
    return probs[:N_NODES, :16]


# speedup vs baseline: 1.2191x; 1.0546x over previous
import functools

import jax
import jax.numpy as jnp
from jax import lax
from jax.experimental import pallas as pl
from jax.experimental.pallas import tpu as pltpu
from jax.experimental.pallas import tpu_sc as plsc

N_NODES = 10000
NPAD = 10240
D_FEAT = 128
N_EDGES = 320000
NW = 32
CH_T = 80
E_PAD = NW * CH_T * 128
PAD_ROW = NPAD - 1
ROWS_T = NPAD // 16

_mesh = plsc.VectorSubcoreMesh(core_axis_name="c", subcore_axis_name="s")


def _deg_body(dst_hbm, ones_hbm, zeros_hbm, out_hbm, idx_v, ones_v, zer_v, deg_sh):
    c = lax.axis_index("c")
    s = lax.axis_index("s")
    wid = s * 2 + c
    pltpu.sync_copy(ones_hbm, ones_v)
    pltpu.sync_copy(zeros_hbm, zer_v)
    pltpu.sync_copy(zer_v, deg_sh.at[pl.ds(s * ROWS_T, ROWS_T)])
    pltpu.sync_copy(dst_hbm.at[wid], idx_v)
    plsc.subcore_barrier()

    def body(j, carry):
        pltpu.sync_copy(ones_v, deg_sh.at[idx_v.at[j]], add=True)
        return carry

    lax.fori_loop(0, CH_T, body, 0)
    plsc.subcore_barrier()
    pltpu.sync_copy(deg_sh.at[pl.ds(s * ROWS_T, ROWS_T)],
                    out_hbm.at[c, pl.ds(s * ROWS_T, ROWS_T)])


def _deg_sc(dstp, ones1, zeros1):
    return pl.kernel(
        _deg_body,
        out_type=jax.ShapeDtypeStruct((2, NPAD), jnp.float32),
        mesh=_mesh,
        scratch_types=[
            pltpu.VMEM((CH_T, 128), jnp.int32),
            pltpu.VMEM((128,), jnp.float32),
            pltpu.VMEM((ROWS_T,), jnp.float32),
            pltpu.VMEM_SHARED((NPAD,), jnp.float32),
        ],
    )(dstp, ones1, zeros1)


NB = 2
NGRP = 4
K_C0 = 16
K_C1 = 4
KMAX = 16


def _edge_loop(src_hbm, dst_hbm, h_hbm, agg_sh,
               sidx, didx, rows, gsem, ssem, s, k):
    def group(g, carry):
        for b in range(NB):
            pltpu.sync_copy(src_hbm.at[s, b, g], sidx[b].at[pl.ds(0, k)])
            pltpu.sync_copy(dst_hbm.at[s, b, g], didx[b].at[pl.ds(0, k)])
            pltpu.async_copy(h_hbm.at[sidx[b].at[0]], rows[b], gsem[b])

        def body(i, carry2):
            scat = []
            for b in range(NB):
                pltpu.make_async_copy(h_hbm.at[sidx[b].at[i]], rows[b],
                                      gsem[b]).wait()
                scat.append(pltpu.async_copy(rows[b],
                                             agg_sh.at[didx[b].at[i]],
                                             ssem[b], add=True))
            for b in range(NB):
                scat[b].wait()
                jn = jnp.minimum(i + 1, k - 1)
                pltpu.async_copy(h_hbm.at[sidx[b].at[jn]], rows[b], gsem[b])
            return carry2

        lax.fori_loop(0, k, body, 0)
        for b in range(NB):
            pltpu.make_async_copy(h_hbm.at[sidx[b].at[k - 1]], rows[b],
                                  gsem[b]).wait()
        return carry

    lax.fori_loop(0, NGRP, group, 0)


def _agg_body(src0_hbm, dst0_hbm, src1_hbm, dst1_hbm, h_hbm, zeros_hbm,
              out_hbm, sidx0, sidx1, didx0, didx1, rows0, rows1, agg_sh,
              gsem0, gsem1, ssem0, ssem1):
    c = lax.axis_index("c")
    s = lax.axis_index("s")
    sidx = (sidx0, sidx1)
    didx = (didx0, didx1)
    rows = (rows0, rows1)
    gsem = (gsem0, gsem1)
    ssem = (ssem0, ssem1)
    pltpu.sync_copy(zeros_hbm, agg_sh.at[pl.ds(s * ROWS_T, ROWS_T)])
    plsc.subcore_barrier()

    @pl.when(c == 0)
    def _():
        _edge_loop(src0_hbm, dst0_hbm, h_hbm, agg_sh,
                   sidx, didx, rows, gsem, ssem, s, K_C0)

    @pl.when(c == 1)
    def _():
        _edge_loop(src1_hbm, dst1_hbm, h_hbm, agg_sh,
                   sidx, didx, rows, gsem, ssem, s, K_C1)

    plsc.subcore_barrier()
    pltpu.sync_copy(agg_sh.at[pl.ds(s * ROWS_T, ROWS_T)],
                    out_hbm.at[c, pl.ds(s * ROWS_T, ROWS_T)])


def _agg_sc(src0, dst0, src1, dst1, h, zeros2):
    return pl.kernel(
        _agg_body,
        out_type=jax.ShapeDtypeStruct((2, NPAD, D_FEAT), jnp.float32),
        mesh=_mesh,
        scratch_types=[
            pltpu.VMEM((KMAX, 128), jnp.int32),
            pltpu.VMEM((KMAX, 128), jnp.int32),
            pltpu.VMEM((KMAX, 128), jnp.int32),
            pltpu.VMEM((KMAX, 128), jnp.int32),
            pltpu.VMEM((128, D_FEAT), jnp.float32),
            pltpu.VMEM((128, D_FEAT), jnp.float32),
            pltpu.VMEM_SHARED((NPAD, D_FEAT), jnp.float32),
            pltpu.SemaphoreType.DMA,
            pltpu.SemaphoreType.DMA,
            pltpu.SemaphoreType.DMA,
            pltpu.SemaphoreType.DMA,
        ],
    )(src0, dst0, src1, dst1, h, zeros2)


def _stage1_body(x_ref, degp_ref, w_ref, h_ref, dinv_ref):
    deg = jnp.sum(degp_ref[...], axis=1, keepdims=True) + 1.0
    dinv = lax.rsqrt(deg)
    dinv_ref[...] = dinv
    h_ref[...] = jnp.dot(x_ref[...], w_ref[...],
                         preferred_element_type=jnp.float32) * dinv


def _stage1_tc(x_pad, degp_t, w1):
    return pl.pallas_call(
        _stage1_body,
        out_shape=(
            jax.ShapeDtypeStruct((NPAD, D_FEAT), jnp.float32),
            jax.ShapeDtypeStruct((NPAD, 1), jnp.float32),
        ),
    )(x_pad, degp_t, w1)


def _stage2_body(agga_ref, aggb_ref, hp_ref, dinv_ref, b_ref, w_ref, out_ref):
    dinv = dinv_ref[...]
    t = (agga_ref[...] + aggb_ref[...] + hp_ref[...]) * dinv + b_ref[...]
    h = jnp.maximum(t, 0.0)
    out_ref[...] = jnp.dot(h, w_ref[...],
                           preferred_element_type=jnp.float32) * dinv


def _stage2_tc(agga, aggb, hp, dinv, b1, w2):
    return pl.pallas_call(
        _stage2_body,
        out_shape=jax.ShapeDtypeStruct((NPAD, D_FEAT), jnp.float32),
    )(agga, aggb, hp, dinv, b1, w2)


def _stage3_body(agga_ref, aggb_ref, hp_ref, dinv_ref, b_ref, w_ref,
                 bmask_ref, out_ref):
    t = (agga_ref[...] + aggb_ref[...] + hp_ref[...]) * dinv_ref[...] + b_ref[...]
    h = jnp.maximum(t, 0.0)
    logits = jnp.dot(h, w_ref[...],
                     preferred_element_type=jnp.float32) + bmask_ref[...]
    m = jnp.max(logits, axis=1, keepdims=True)
    e = jnp.exp(logits - m)
    out_ref[...] = e / jnp.sum(e, axis=1, keepdims=True)


def _stage3_tc(agga, aggb, hp, dinv, b2, wfc_pad, bmask):
    return pl.pallas_call(
        _stage3_body,
        out_shape=jax.ShapeDtypeStruct((NPAD, 128), jnp.float32),
    )(agga, aggb, hp, dinv, b2, wfc_pad, bmask)


def kernel(x, edge_index, W1, b1, W2, b2, Wfc, bfc):
    x_pad = jnp.pad(x, ((0, NPAD - N_NODES), (0, 0)))
    epad = jnp.full((E_PAD - N_EDGES,), PAD_ROW, jnp.int32)

    n0 = 16 * NB * NGRP * K_C0 * 128

    def _prep(idx):
        a = jnp.concatenate([idx, epad])
        a0 = a[:n0].reshape(16, NGRP, K_C0, NB, 128).transpose(0, 3, 1, 2, 4)
        a1 = a[n0:].reshape(16, NGRP, K_C1, NB, 128).transpose(0, 3, 1, 2, 4)
        return a0, a1

    src0, src1 = _prep(edge_index[0])
    dst0, dst1 = _prep(edge_index[1])
    dstp_flat = jnp.concatenate([edge_index[1], epad]).reshape(NW, CH_T, 128)

    ones1 = jnp.ones((128,), jnp.float32)
    zeros1 = jnp.zeros((ROWS_T,), jnp.float32)
    zeros2 = jnp.zeros((ROWS_T, D_FEAT), jnp.float32)
    b1r = b1.reshape(1, D_FEAT)
    b2r = b2.reshape(1, D_FEAT)
    wfc_pad = jnp.pad(Wfc, ((0, 0), (0, 128 - Wfc.shape[1])))
    bmask = jnp.concatenate(
        [bfc, jnp.full((128 - bfc.shape[0],), -1e30, jnp.float32)]).reshape(1, 128)

    degp = _deg_sc(dstp_flat, ones1, zeros1)
    degp_t = degp.T

    h1p, dinv = _stage1_tc(x_pad, degp_t, W1)
    agg1 = _agg_sc(src0, dst0, src1, dst1, h1p, zeros2)
    h2p = _stage2_tc(agg1[0], agg1[1], h1p, dinv, b1r, W2)
    agg2 = _agg_sc(src0, dst0, src1, dst1, h2p, zeros2)
    probs = _stage3_tc(agg2[0], agg2[1], h2p, dinv, b2r, wfc_pad, bmask)
    return probs[:N_NODES, :16]


# speedup vs baseline: 1.2218x; 1.0022x over previous
import functools

import jax
import jax.numpy as jnp
from jax import lax
from jax.experimental import pallas as pl
from jax.experimental.pallas import tpu as pltpu
from jax.experimental.pallas import tpu_sc as plsc

N_NODES = 10000
NPAD = 10240
D_FEAT = 128
N_EDGES = 320000
NW = 32
CH_T = 80
E_PAD = NW * CH_T * 128
PAD_ROW = NPAD - 1
ROWS_T = NPAD // 16

_mesh = plsc.VectorSubcoreMesh(core_axis_name="c", subcore_axis_name="s")


def _deg_body(dst_hbm, ones_hbm, zeros_hbm, out_hbm, idx_v, ones_v, zer_v, deg_sh):
    c = lax.axis_index("c")
    s = lax.axis_index("s")
    wid = s * 2 + c
    pltpu.sync_copy(ones_hbm, ones_v)
    pltpu.sync_copy(zeros_hbm, zer_v)
    pltpu.sync_copy(zer_v, deg_sh.at[pl.ds(s * ROWS_T, ROWS_T)])
    pltpu.sync_copy(dst_hbm.at[wid], idx_v)
    plsc.subcore_barrier()

    def body(j, carry):
        pltpu.sync_copy(ones_v, deg_sh.at[idx_v.at[j]], add=True)
        return carry

    lax.fori_loop(0, CH_T, body, 0)
    plsc.subcore_barrier()
    pltpu.sync_copy(deg_sh.at[pl.ds(s * ROWS_T, ROWS_T)],
                    out_hbm.at[c, pl.ds(s * ROWS_T, ROWS_T)])


def _deg_sc(dstp, ones1, zeros1):
    return pl.kernel(
        _deg_body,
        out_type=jax.ShapeDtypeStruct((2, NPAD), jnp.float32),
        mesh=_mesh,
        scratch_types=[
            pltpu.VMEM((CH_T, 128), jnp.int32),
            pltpu.VMEM((128,), jnp.float32),
            pltpu.VMEM((ROWS_T,), jnp.float32),
            pltpu.VMEM_SHARED((NPAD,), jnp.float32),
        ],
    )(dstp, ones1, zeros1)


NB = 2
NGRP = 4
K_C0 = 13
K_C1 = 7
KMAX = 16


def _edge_loop(src_hbm, dst_hbm, h_hbm, agg_sh,
               sidx, didx, rows, gsem, ssem, s, k):
    def group(g, carry):
        for b in range(NB):
            pltpu.sync_copy(src_hbm.at[s, b, g], sidx[b].at[pl.ds(0, k)])
            pltpu.sync_copy(dst_hbm.at[s, b, g], didx[b].at[pl.ds(0, k)])
            pltpu.async_copy(h_hbm.at[sidx[b].at[0]], rows[b], gsem[b])

        def body(i, carry2):
            scat = []
            for b in range(NB):
                pltpu.make_async_copy(h_hbm.at[sidx[b].at[i]], rows[b],
                                      gsem[b]).wait()
                scat.append(pltpu.async_copy(rows[b],
                                             agg_sh.at[didx[b].at[i]],
                                             ssem[b], add=True))
            for b in range(NB):
                scat[b].wait()
                jn = jnp.minimum(i + 1, k - 1)
                pltpu.async_copy(h_hbm.at[sidx[b].at[jn]], rows[b], gsem[b])
            return carry2

        lax.fori_loop(0, k, body, 0)
        for b in range(NB):
            pltpu.make_async_copy(h_hbm.at[sidx[b].at[k - 1]], rows[b],
                                  gsem[b]).wait()
        return carry

    lax.fori_loop(0, NGRP, group, 0)


def _agg_body(src0_hbm, dst0_hbm, src1_hbm, dst1_hbm, h_hbm, zeros_hbm,
              out_hbm, sidx0, sidx1, didx0, didx1, rows0, rows1, agg_sh,
              gsem0, gsem1, ssem0, ssem1):
    c = lax.axis_index("c")
    s = lax.axis_index("s")
    sidx = (sidx0, sidx1)
    didx = (didx0, didx1)
    rows = (rows0, rows1)
    gsem = (gsem0, gsem1)
    ssem = (ssem0, ssem1)
    pltpu.sync_copy(zeros_hbm, agg_sh.at[pl.ds(s * ROWS_T, ROWS_T)])
    plsc.subcore_barrier()

    @pl.when(c == 0)
    def _():
        _edge_loop(src0_hbm, dst0_hbm, h_hbm, agg_sh,
                   sidx, didx, rows, gsem, ssem, s, K_C0)

    @pl.when(c == 1)
    def _():
        _edge_loop(src1_hbm, dst1_hbm, h_hbm, agg_sh,
                   sidx, didx, rows, gsem, ssem, s, K_C1)

    plsc.subcore_barrier()
    pltpu.sync_copy(agg_sh.at[pl.ds(s * ROWS_T, ROWS_T)],
                    out_hbm.at[c, pl.ds(s * ROWS_T, ROWS_T)])


def _agg_sc(src0, dst0, src1, dst1, h, zeros2):
    return pl.kernel(
        _agg_body,
        out_type=jax.ShapeDtypeStruct((2, NPAD, D_FEAT), jnp.float32),
        mesh=_mesh,
        scratch_types=[
            pltpu.VMEM((KMAX, 128), jnp.int32),
            pltpu.VMEM((KMAX, 128), jnp.int32),
            pltpu.VMEM((KMAX, 128), jnp.int32),
            pltpu.VMEM((KMAX, 128), jnp.int32),
            pltpu.VMEM((128, D_FEAT), jnp.float32),
            pltpu.VMEM((128, D_FEAT), jnp.float32),
            pltpu.VMEM_SHARED((NPAD, D_FEAT), jnp.float32),
            pltpu.SemaphoreType.DMA,
            pltpu.SemaphoreType.DMA,
            pltpu.SemaphoreType.DMA,
            pltpu.SemaphoreType.DMA,
        ],
    )(src0, dst0, src1, dst1, h, zeros2)


def _stage1_body(x_ref, degp_ref, w_ref, h_ref, dinv_ref):
    deg = jnp.sum(degp_ref[...], axis=1, keepdims=True) + 1.0
    dinv = lax.rsqrt(deg)
    dinv_ref[...] = dinv
    h_ref[...] = jnp.dot(x_ref[...], w_ref[...],
                         preferred_element_type=jnp.float32) * dinv


def _stage1_tc(x_pad, degp_t, w1):
    return pl.pallas_call(
        _stage1_body,
        out_shape=(
            jax.ShapeDtypeStruct((NPAD, D_FEAT), jnp.float32),
            jax.ShapeDtypeStruct((NPAD, 1), jnp.float32),
        ),
    )(x_pad, degp_t, w1)


def _stage2_body(agga_ref, aggb_ref, hp_ref, dinv_ref, b_ref, w_ref, out_ref):
    dinv = dinv_ref[...]
    t = (agga_ref[...] + aggb_ref[...] + hp_ref[...]) * dinv + b_ref[...]
    h = jnp.maximum(t, 0.0)
    out_ref[...] = jnp.dot(h, w_ref[...],
                           preferred_element_type=jnp.float32) * dinv


def _stage2_tc(agga, aggb, hp, dinv, b1, w2):
    return pl.pallas_call(
        _stage2_body,
        out_shape=jax.ShapeDtypeStruct((NPAD, D_FEAT), jnp.float32),
    )(agga, aggb, hp, dinv, b1, w2)


def _stage3_body(agga_ref, aggb_ref, hp_ref, dinv_ref, b_ref, w_ref,
                 bmask_ref, out_ref):
    t = (agga_ref[...] + aggb_ref[...] + hp_ref[...]) * dinv_ref[...] + b_ref[...]
    h = jnp.maximum(t, 0.0)
    logits = jnp.dot(h, w_ref[...],
                     preferred_element_type=jnp.float32) + bmask_ref[...]
    m = jnp.max(logits, axis=1, keepdims=True)
    e = jnp.exp(logits - m)
    out_ref[...] = e / jnp.sum(e, axis=1, keepdims=True)


def _stage3_tc(agga, aggb, hp, dinv, b2, wfc_pad, bmask):
    return pl.pallas_call(
        _stage3_body,
        out_shape=jax.ShapeDtypeStruct((NPAD, 128), jnp.float32),
    )(agga, aggb, hp, dinv, b2, wfc_pad, bmask)


def kernel(x, edge_index, W1, b1, W2, b2, Wfc, bfc):
    x_pad = jnp.pad(x, ((0, NPAD - N_NODES), (0, 0)))
    epad = jnp.full((E_PAD - N_EDGES,), PAD_ROW, jnp.int32)

    n0 = 16 * NB * NGRP * K_C0 * 128

    def _prep(idx):
        a = jnp.concatenate([idx, epad])
        a0 = a[:n0].reshape(16, NGRP, K_C0, NB, 128).transpose(0, 3, 1, 2, 4)
        a1 = a[n0:].reshape(16, NGRP, K_C1, NB, 128).transpose(0, 3, 1, 2, 4)
        return a0, a1

    src0, src1 = _prep(edge_index[0])
    dst0, dst1 = _prep(edge_index[1])
    dstp_flat = jnp.concatenate([edge_index[1], epad]).reshape(NW, CH_T, 128)

    ones1 = jnp.ones((128,), jnp.float32)
    zeros1 = jnp.zeros((ROWS_T,), jnp.float32)
    zeros2 = jnp.zeros((ROWS_T, D_FEAT), jnp.float32)
    b1r = b1.reshape(1, D_FEAT)
    b2r = b2.reshape(1, D_FEAT)
    wfc_pad = jnp.pad(Wfc, ((0, 0), (0, 128 - Wfc.shape[1])))
    bmask = jnp.concatenate(
        [bfc, jnp.full((128 - bfc.shape[0],), -1e30, jnp.float32)]).reshape(1, 128)

    degp = _deg_sc(dstp_flat, ones1, zeros1)
    degp_t = degp.T

    h1p, dinv = _stage1_tc(x_pad, degp_t, W1)
    agg1 = _agg_sc(src0, dst0, src1, dst1, h1p, zeros2)
    h2p = _stage2_tc(agg1[0], agg1[1], h1p, dinv, b1r, W2)
    agg2 = _agg_sc(src0, dst0, src1, dst1, h2p, zeros2)
    probs = _stage3_tc(agg2[0], agg2[1], h2p, dinv, b2r, wfc_pad, bmask)
    return probs[:N_NODES, :16]


# speedup vs baseline: 1.2905x; 1.0562x over previous
import functools

import jax
import jax.numpy as jnp
from jax import lax
from jax.experimental import pallas as pl
from jax.experimental.pallas import tpu as pltpu
from jax.experimental.pallas import tpu_sc as plsc

N_NODES = 10000
NPAD = 10240
D_FEAT = 128
N_EDGES = 320000
NW = 32
CH_T = 80
E_PAD = NW * CH_T * 128
PAD_ROW = NPAD - 1
ROWS_T = NPAD // 16

_mesh = plsc.VectorSubcoreMesh(core_axis_name="c", subcore_axis_name="s")


def _deg_body(dst_hbm, ones_hbm, zeros_hbm, out_hbm, idx_v, ones_v, zer_v, deg_sh):
    c = lax.axis_index("c")
    s = lax.axis_index("s")
    wid = s * 2 + c
    pltpu.sync_copy(ones_hbm, ones_v)
    pltpu.sync_copy(zeros_hbm, zer_v)
    pltpu.sync_copy(zer_v, deg_sh.at[pl.ds(s * ROWS_T, ROWS_T)])
    pltpu.sync_copy(dst_hbm.at[wid], idx_v)
    plsc.subcore_barrier()

    def body(j, carry):
        pltpu.sync_copy(ones_v, deg_sh.at[idx_v.at[j]], add=True)
        return carry

    lax.fori_loop(0, CH_T, body, 0)
    plsc.subcore_barrier()
    pltpu.sync_copy(deg_sh.at[pl.ds(s * ROWS_T, ROWS_T)],
                    out_hbm.at[c, pl.ds(s * ROWS_T, ROWS_T)])


def _deg_sc(dstp, ones1, zeros1):
    return pl.kernel(
        _deg_body,
        out_type=jax.ShapeDtypeStruct((2, NPAD), jnp.float32),
        mesh=_mesh,
        scratch_types=[
            pltpu.VMEM((CH_T, 128), jnp.int32),
            pltpu.VMEM((128,), jnp.float32),
            pltpu.VMEM((ROWS_T,), jnp.float32),
            pltpu.VMEM_SHARED((NPAD,), jnp.float32),
        ],
    )(dstp, ones1, zeros1)


NB = 2
NGRP = 2
K_C0 = 26
K_C1 = 14
KMAX = 26


def _edge_loop(src_hbm, dst_hbm, h_hbm, agg_sh,
               sidx, didx, rows, gsem, ssem, s, k):
    def group(g, carry):
        for b in range(NB):
            pltpu.sync_copy(src_hbm.at[s, b, g], sidx[b].at[pl.ds(0, k)])
            pltpu.sync_copy(dst_hbm.at[s, b, g], didx[b].at[pl.ds(0, k)])
            pltpu.async_copy(h_hbm.at[sidx[b].at[0]], rows[b], gsem[b])

        def body(i, carry2):
            scat = []
            for b in range(NB):
                pltpu.make_async_copy(h_hbm.at[sidx[b].at[i]], rows[b],
                                      gsem[b]).wait()
                scat.append(pltpu.async_copy(rows[b],
                                             agg_sh.at[didx[b].at[i]],
                                             ssem[b], add=True))
            for b in range(NB):
                scat[b].wait()
                jn = jnp.minimum(i + 1, k - 1)
                pltpu.async_copy(h_hbm.at[sidx[b].at[jn]], rows[b], gsem[b])
            return carry2

        lax.fori_loop(0, k, body, 0)
        for b in range(NB):
            pltpu.make_async_copy(h_hbm.at[sidx[b].at[k - 1]], rows[b],
                                  gsem[b]).wait()
        return carry

    lax.fori_loop(0, NGRP, group, 0)


def _agg_body(src0_hbm, dst0_hbm, src1_hbm, dst1_hbm, h_hbm, zeros_hbm,
              out_hbm, sidx0, sidx1, didx0, didx1, rows0, rows1, agg_sh,
              gsem0, gsem1, ssem0, ssem1):
    c = lax.axis_index("c")
    s = lax.axis_index("s")
    sidx = (sidx0, sidx1)
    didx = (didx0, didx1)
    rows = (rows0, rows1)
    gsem = (gsem0, gsem1)
    ssem = (ssem0, ssem1)
    pltpu.sync_copy(zeros_hbm, agg_sh.at[pl.ds(s * ROWS_T, ROWS_T)])
    plsc.subcore_barrier()

    @pl.when(c == 0)
    def _():
        _edge_loop(src0_hbm, dst0_hbm, h_hbm, agg_sh,
                   sidx, didx, rows, gsem, ssem, s, K_C0)

    @pl.when(c == 1)
    def _():
        _edge_loop(src1_hbm, dst1_hbm, h_hbm, agg_sh,
                   sidx, didx, rows, gsem, ssem, s, K_C1)

    plsc.subcore_barrier()
    pltpu.sync_copy(agg_sh.at[pl.ds(s * ROWS_T, ROWS_T)],
                    out_hbm.at[c, pl.ds(s * ROWS_T, ROWS_T)])


def _agg_sc(src0, dst0, src1, dst1, h, zeros2):
    return pl.kernel(
        _agg_body,
        out_type=jax.ShapeDtypeStruct((2, NPAD, D_FEAT), jnp.float32),
        mesh=_mesh,
        scratch_types=[
            pltpu.VMEM((KMAX, 128), jnp.int32),
            pltpu.VMEM((KMAX, 128), jnp.int32),
            pltpu.VMEM((KMAX, 128), jnp.int32),
            pltpu.VMEM((KMAX, 128), jnp.int32),
            pltpu.VMEM((128, D_FEAT), jnp.float32),
            pltpu.VMEM((128, D_FEAT), jnp.float32),
            pltpu.VMEM_SHARED((NPAD, D_FEAT), jnp.float32),
            pltpu.SemaphoreType.DMA,
            pltpu.SemaphoreType.DMA,
            pltpu.SemaphoreType.DMA,
            pltpu.SemaphoreType.DMA,
        ],
    )(src0, dst0, src1, dst1, h, zeros2)


def _stage1_body(x_ref, degp_ref, w_ref, h_ref, dinv_ref):
    deg = jnp.sum(degp_ref[...], axis=1, keepdims=True) + 1.0
    dinv = lax.rsqrt(deg)
    dinv_ref[...] = dinv
    h_ref[...] = jnp.dot(x_ref[...], w_ref[...],
                         preferred_element_type=jnp.float32) * dinv


def _stage1_tc(x_pad, degp_t, w1):
    return pl.pallas_call(
        _stage1_body,
        out_shape=(
            jax.ShapeDtypeStruct((NPAD, D_FEAT), jnp.float32),
            jax.ShapeDtypeStruct((NPAD, 1), jnp.float32),
        ),
    )(x_pad, degp_t, w1)


def _stage2_body(agga_ref, aggb_ref, hp_ref, dinv_ref, b_ref, w_ref, out_ref):
    dinv = dinv_ref[...]
    t = (agga_ref[...] + aggb_ref[...] + hp_ref[...]) * dinv + b_ref[...]
    h = jnp.maximum(t, 0.0)
    out_ref[...] = jnp.dot(h, w_ref[...],
                           preferred_element_type=jnp.float32) * dinv


def _stage2_tc(agga, aggb, hp, dinv, b1, w2):
    return pl.pallas_call(
        _stage2_body,
        out_shape=jax.ShapeDtypeStruct((NPAD, D_FEAT), jnp.float32),
    )(agga, aggb, hp, dinv, b1, w2)


def _stage3_body(agga_ref, aggb_ref, hp_ref, dinv_ref, b_ref, w_ref,
                 bmask_ref, out_ref):
    t = (agga_ref[...] + aggb_ref[...] + hp_ref[...]) * dinv_ref[...] + b_ref[...]
    h = jnp.maximum(t, 0.0)
    logits = jnp.dot(h, w_ref[...],
                     preferred_element_type=jnp.float32) + bmask_ref[...]
    m = jnp.max(logits, axis=1, keepdims=True)
    e = jnp.exp(logits - m)
    out_ref[...] = e / jnp.sum(e, axis=1, keepdims=True)


def _stage3_tc(agga, aggb, hp, dinv, b2, wfc_pad, bmask):
    return pl.pallas_call(
        _stage3_body,
        out_shape=jax.ShapeDtypeStruct((NPAD, 128), jnp.float32),
    )(agga, aggb, hp, dinv, b2, wfc_pad, bmask)


def kernel(x, edge_index, W1, b1, W2, b2, Wfc, bfc):
    x_pad = jnp.pad(x, ((0, NPAD - N_NODES), (0, 0)))
    epad = jnp.full((E_PAD - N_EDGES,), PAD_ROW, jnp.int32)

    n0 = 16 * NB * NGRP * K_C0 * 128

    def _prep(idx):
        a = jnp.concatenate([idx, epad])
        a0 = a[:n0].reshape(16, NGRP, K_C0, NB, 128).transpose(0, 3, 1, 2, 4)
        a1 = a[n0:].reshape(16, NGRP, K_C1, NB, 128).transpose(0, 3, 1, 2, 4)
        return a0, a1

    src0, src1 = _prep(edge_index[0])
    dst0, dst1 = _prep(edge_index[1])
    dstp_flat = jnp.concatenate([edge_index[1], epad]).reshape(NW, CH_T, 128)

    ones1 = jnp.ones((128,), jnp.float32)
    zeros1 = jnp.zeros((ROWS_T,), jnp.float32)
    zeros2 = jnp.zeros((ROWS_T, D_FEAT), jnp.float32)
    b1r = b1.reshape(1, D_FEAT)
    b2r = b2.reshape(1, D_FEAT)
    wfc_pad = jnp.pad(Wfc, ((0, 0), (0, 128 - Wfc.shape[1])))
    bmask = jnp.concatenate(
        [bfc, jnp.full((128 - bfc.shape[0],), -1e30, jnp.float32)]).reshape(1, 128)

    degp = _deg_sc(dstp_flat, ones1, zeros1)
    degp_t = degp.T

    h1p, dinv = _stage1_tc(x_pad, degp_t, W1)
    agg1 = _agg_sc(src0, dst0, src1, dst1, h1p, zeros2)
    h2p = _stage2_tc(agg1[0], agg1[1], h1p, dinv, b1r, W2)
    agg2 = _agg_sc(src0, dst0, src1, dst1, h2p, zeros2)
    probs = _stage3_tc(agg2[0], agg2[1], h2p, dinv, b2r, wfc_pad, bmask)
    return probs[:N_NODES, :16]
